# Initial kernel scaffold; baseline (speedup 1.0000x reference)
import jax
import jax.numpy as jnp
from jax.experimental import pallas as pl


def kernel(x, edge_index, W1, b1, W2, b2, W_out, b_out):
    raise NotImplementedError("write your pallas kernel here")


# speedup vs baseline: 2.9233x; 2.9233x over previous
import functools

import jax
import jax.numpy as jnp
from jax import lax
from jax.experimental import pallas as pl
from jax.experimental.pallas import tpu as pltpu
from jax.experimental.pallas import tpu_sc as plsc

N = 10000
NP = 10240
E = 160000
NC, NS, L = 2, 16, 16
NW = NC * NS
EPT = 5120
EP = NW * EPT
D_IN = 256
D_HID = 512
DQ = D_HID // 4
CHUNK = NP // NC
CAP = 5248
GB = 128
CACC = CHUNK + 8
ZROWS = CHUNK // NS

_mesh = plsc.VectorSubcoreMesh(
    core_axis_name="c", subcore_axis_name="s", num_cores=NC, num_subcores=NS)
_sc_params = pltpu.CompilerParams(needs_layout_passes=False)


HNP = NP // 2


@functools.partial(
    pl.kernel,
    out_type=(
        jax.ShapeDtypeStruct((NW, NP), jnp.float32),
        jax.ShapeDtypeStruct((NC, NW, CAP), jnp.int32),
        jax.ShapeDtypeStruct((NC, NW, CAP), jnp.int32),
        jax.ShapeDtypeStruct((NW * 16,), jnp.int32),
    ),
    mesh=_mesh,
    scratch_types=dict(
        srcv=pltpu.VMEM((EPT,), jnp.int32),
        dstv=pltpu.VMEM((EPT,), jnp.int32),
        h16=pltpu.VMEM((16, HNP), jnp.float32),
        hred=pltpu.VMEM((HNP,), jnp.float32),
        bs=[pltpu.VMEM((CAP,), jnp.int32) for _ in range(NC)],
        bd=[pltpu.VMEM((CAP,), jnp.int32) for _ in range(NC)],
        cntv=pltpu.VMEM((16,), jnp.int32),
    ),
    compiler_params=_sc_params,
)
def _sc_prep(src_hbm, dst_hbm, hist_hbm, bsrc_hbm, bdst_hbm, bcnt_hbm, *,
             srcv, dstv, h16, hred, bs, bd, cntv):
    c = lax.axis_index("c")
    s = lax.axis_index("s")
    wid = s * NC + c
    base = wid * EPT

    pltpu.sync_copy(src_hbm.at[pl.ds(base, EPT)], srcv)
    pltpu.sync_copy(dst_hbm.at[pl.ds(base, EPT)], dstv)

    lane = lax.iota(jnp.int32, L)
    onesv = jnp.ones((L,), jnp.float32)
    for half in range(2):
        def hz_step(i, _):
            for r in range(16):
                h16[r, pl.ds(i * L, L)] = jnp.zeros((L,), jnp.float32)
            return 0
        lax.fori_loop(0, HNP // L, hz_step, 0)

        def hacc_step(i, _):
            dv = dstv[pl.ds(i * L, L)]
            m = (dv >= half * HNP) & (dv < (half + 1) * HNP)
            plsc.addupdate_scatter(h16, [lane, dv - half * HNP], onesv,
                                   mask=m)
            return 0
        lax.fori_loop(0, EPT // L, hacc_step, 0)

        def hr_step(i, _):
            acc = h16[0, pl.ds(i * L, L)]
            for r in range(1, 16):
                acc = acc + h16[r, pl.ds(i * L, L)]
            hred[pl.ds(i * L, L)] = acc
            return 0
        lax.fori_loop(0, HNP // L, hr_step, 0)
        pltpu.sync_copy(hred,
                        hist_hbm.at[wid].at[pl.ds(half * HNP, HNP)])

    def pre_step(i, _):
        for ch in range(NC):
            bs[ch][pl.ds(i * L, L)] = jnp.zeros((L,), jnp.int32)
            bd[ch][pl.ds(i * L, L)] = jnp.full((L,), CHUNK, jnp.int32)
        return 0
    lax.fori_loop(0, CAP // L, pre_step, 0)

    def comp_step(i, cnts):
        sv = srcv[pl.ds(i * L, L)]
        dv = dstv[pl.ds(i * L, L)]
        out = []
        for ch in range(NC):
            m = (dv >= ch * CHUNK) & (dv < (ch + 1) * CHUNK)
            inc = m.astype(jnp.int32)
            pos = plsc.cumsum(inc) - 1 + cnts[ch]
            plsc.store_scatter(bs[ch], [pos], sv, mask=m)
            plsc.store_scatter(bd[ch], [pos], dv - ch * CHUNK, mask=m)
            out.append(pos[15] + 1)
        return tuple(out)
    zero = jnp.zeros((), jnp.int32)
    cnts = lax.fori_loop(0, EPT // L, comp_step, (zero,) * NC)

    cv = jnp.zeros((L,), jnp.int32)
    for ch in range(NC):
        cv = jnp.where(lane == ch, cnts[ch], cv)
        pltpu.sync_copy(bs[ch], bsrc_hbm.at[ch, wid])
        pltpu.sync_copy(bd[ch], bdst_hbm.at[ch, wid])
    npadv = lax.bitwise_and(cv + 127, jnp.full((L,), -128, jnp.int32))
    cntv[...] = npadv
    pltpu.sync_copy(cntv, bcnt_hbm.at[pl.ds(wid * 16, 16)])


@functools.partial(
    pl.kernel,
    out_type=tuple(
        jax.ShapeDtypeStruct((NP, DQ), jnp.float32) for _ in range(4)),
    mesh=_mesh,
    scratch_types=dict(
        si=pltpu.VMEM((CAP,), jnp.int32),
        sq=pltpu.VMEM((CAP,), jnp.int32),
        dl=pltpu.VMEM((CAP,), jnp.int32),
        rows0=pltpu.VMEM((GB, DQ), jnp.float32),
        rows1=pltpu.VMEM((GB, DQ), jnp.float32),
        zbuf=pltpu.VMEM((16, DQ), jnp.float32),
        acc=pltpu.VMEM_SHARED((CACC, DQ), jnp.float32),
        bcnt_sp=pltpu.VMEM_SHARED((NW * 16,), jnp.int32),
        cnt_sm=pltpu.SMEM((NW * 16,), jnp.int32),
        sem=pltpu.SemaphoreType.DMA,
    ),
    compiler_params=_sc_params,
)
def _sc_agg(hs_hbm, bsrc_hbm, bdst_hbm, bcnt_hbm,
            agg0_hbm, agg1_hbm, agg2_hbm, agg3_hbm, *, si, sq, dl,
            rows0, rows1, zbuf, acc, bcnt_sp, cnt_sm, sem):
    c = lax.axis_index("c")
    s = lax.axis_index("s")
    aggs = (agg0_hbm, agg1_hbm, agg2_hbm, agg3_hbm)

    @pl.when(s == 0)
    def _():
        pltpu.sync_copy(bcnt_hbm, bcnt_sp)
    plsc.subcore_barrier()
    pltpu.sync_copy(bcnt_sp, cnt_sm)

    def zb_step(i, _):
        for r in range(16):
            zbuf[r, pl.ds(i * L, L)] = jnp.zeros((L,), jnp.float32)
        return 0
    lax.fori_loop(0, DQ // L, zb_step, 0)

    for q in range(4):
        for k in range(ZROWS // 16):
            pltpu.sync_copy(zbuf, acc.at[pl.ds(s * ZROWS + 16 * k, 16)])
        plsc.subcore_barrier()

        for sub in range(2):
            lid = s * 2 + sub
            pltpu.sync_copy(bsrc_hbm.at[c, lid], si)
            pltpu.sync_copy(bdst_hbm.at[c, lid], dl)
            n_pad = cnt_sm[lid * 16 + c]
            nb = n_pad // GB

            def sq_step(i, _):
                sq[pl.ds(i * L, L)] = si[pl.ds(i * L, L)] * 4 + q
                return 0
            lax.fori_loop(0, CAP // L, sq_step, 0)

            def body(g, _):
                d0 = pltpu.async_copy(
                    hs_hbm.at[sq.at[pl.ds(2 * g * GB, GB)]], rows0, sem)
                d1 = pltpu.async_copy(
                    hs_hbm.at[sq.at[pl.ds((2 * g + 1) * GB, GB)]],
                    rows1, sem)
                d0.wait()
                pltpu.sync_copy(
                    rows0, acc.at[dl.at[pl.ds(2 * g * GB, GB)]], add=True)
                d1.wait()
                pltpu.sync_copy(
                    rows1, acc.at[dl.at[pl.ds((2 * g + 1) * GB, GB)]],
                    add=True)
                return 0
            lax.fori_loop(0, nb // 2, body, 0)

            @pl.when(nb % 2 == 1)
            def _():
                g0 = nb - 1
                d = pltpu.async_copy(
                    hs_hbm.at[sq.at[pl.ds(g0 * GB, GB)]], rows0, sem)
                d.wait()
                pltpu.sync_copy(
                    rows0, acc.at[dl.at[pl.ds(g0 * GB, GB)]], add=True)

        plsc.subcore_barrier()
        pltpu.sync_copy(
            acc.at[pl.ds(s * ZROWS, ZROWS)],
            aggs[q].at[pl.ds(c * CHUNK + s * ZROWS, ZROWS)])


ROWS_B = 256
GRID = NP // ROWS_B


def _dis_from_hist(hist_blk):
    deg = jnp.sum(hist_blk, axis=1, keepdims=True) + 1.0
    return lax.rsqrt(deg)


def _tc_a_body(x_ref, w_ref, hist_ref, o_ref):
    dis = _dis_from_hist(hist_ref[...])
    h = jnp.dot(x_ref[...], w_ref[...], preferred_element_type=jnp.float32)
    o_ref[...] = h * dis


def _z_from_quarters(agg_refs, hs_ref, dis, b):
    agg = jnp.concatenate([a[...] for a in agg_refs], axis=1)
    return (agg + hs_ref[...]) * dis + b


def _tc_b_body(a0, a1, a2, a3, hs_ref, hist_ref, b_ref, w_ref, o_ref):
    dis = _dis_from_hist(hist_ref[...])
    z = _z_from_quarters((a0, a1, a2, a3), hs_ref, dis, b_ref[...])
    z = jnp.maximum(z, 0.0)
    h = jnp.dot(z, w_ref[...], preferred_element_type=jnp.float32)
    o_ref[...] = h * dis


def _tc_c_body(a0, a1, a2, a3, hs_ref, hist_ref, b_ref, w_ref, bo_ref,
               o_ref):
    dis = _dis_from_hist(hist_ref[...])
    z = _z_from_quarters((a0, a1, a2, a3), hs_ref, dis, b_ref[...])
    o_ref[...] = (
        jnp.dot(z, w_ref[...], preferred_element_type=jnp.float32)
        + bo_ref[...])


def _row_spec(cols):
    return pl.BlockSpec((ROWS_B, cols), lambda i: (i, 0))


_hist_spec = pl.BlockSpec((ROWS_B, NW), lambda i: (i, 0))


def _full_spec(r, cols):
    return pl.BlockSpec((r, cols), lambda i: (0, 0))


_tc_a = pl.pallas_call(
    _tc_a_body,
    grid=(GRID,),
    in_specs=[_row_spec(D_IN), _full_spec(D_IN, D_HID), _hist_spec],
    out_specs=_row_spec(D_HID),
    out_shape=jax.ShapeDtypeStruct((NP, D_HID), jnp.float32),
)

_tc_b = pl.pallas_call(
    _tc_b_body,
    grid=(GRID,),
    in_specs=[_row_spec(DQ)] * 4 + [
        _row_spec(D_HID), _hist_spec,
        _full_spec(1, D_HID), _full_spec(D_HID, D_HID)],
    out_specs=_row_spec(D_HID),
    out_shape=jax.ShapeDtypeStruct((NP, D_HID), jnp.float32),
)

_tc_c = pl.pallas_call(
    _tc_c_body,
    grid=(GRID,),
    in_specs=[_row_spec(DQ)] * 4 + [
        _row_spec(D_HID), _hist_spec,
        _full_spec(1, D_HID), _full_spec(D_HID, 128),
        _full_spec(1, 128)],
    out_specs=_row_spec(128),
    out_shape=jax.ShapeDtypeStruct((NP, 128), jnp.float32),
)


def kernel(x, edge_index, W1, b1, W2, b2, W_out, b_out):
    src = edge_index[0].astype(jnp.int32)
    dst = edge_index[1].astype(jnp.int32)
    pad = EP - E
    src_p = jnp.concatenate([src, jnp.zeros((pad,), jnp.int32)])
    dst_p = jnp.concatenate([dst, jnp.full((pad,), NP - 1, jnp.int32)])
    x_p = jnp.concatenate(
        [x, jnp.zeros((NP - N, D_IN), jnp.float32)], axis=0)

    hist, bsrc, bdst, bcnt = _sc_prep(src_p, dst_p)
    hist2 = hist.T

    hs1 = _tc_a(x_p, W1, hist2)
    agg1 = _sc_agg(hs1.reshape(NP * 4, DQ), bsrc, bdst, bcnt)
    hs2 = _tc_b(*agg1, hs1, hist2, b1.reshape(1, D_HID), W2)
    agg2 = _sc_agg(hs2.reshape(NP * 4, DQ), bsrc, bdst, bcnt)
    wo = jnp.pad(W_out, ((0, 0), (0, 128 - W_out.shape[1])))
    bo = jnp.pad(b_out, (0, 128 - b_out.shape[0])).reshape(1, 128)
    out = _tc_c(*agg2, hs2, hist2, b2.reshape(1, D_HID), wo, bo)
    return out[:N, :W_out.shape[1]]
